# baseline (speedup 1.0000x reference)
import functools

import jax
import jax.numpy as jnp
from jax import lax
from jax.experimental import pallas as pl
from jax.experimental.pallas import tpu as pltpu
from jax.experimental.pallas import tpu_sc as plsc

E = 8
H = 1024
FF = 1408
SFF = 2816
T = 2048
TILE_M = 256
M_T = T // TILE_M
G_TILES = 24
P_MAX = G_TILES * TILE_M
NW = 32
TPW = T // NW


def _router_kernel(x_ref, gw_ref, sg_ref,
                   pos0_ref, pos1_ref, wcomb_ref, tile_e_ref,
                   cnt_ref):
    p = pl.program_id(0)
    m = pl.program_id(1)

    xf = x_ref[...]
    logits = jnp.dot(xf, gw_ref[...].T, preferred_element_type=jnp.float32)
    prob = jax.nn.softmax(logits, axis=-1)
    lanes = lax.broadcasted_iota(jnp.int32, (TILE_M, E), 1)
    m1 = jnp.max(prob, axis=-1, keepdims=True)
    i1 = jnp.min(jnp.where(prob == m1, lanes, E), axis=-1, keepdims=True)
    sel1 = lanes == i1
    pm = jnp.where(sel1, -jnp.inf, prob)
    m2 = jnp.max(pm, axis=-1, keepdims=True)
    i2 = jnp.min(jnp.where(pm == m2, lanes, E), axis=-1, keepdims=True)
    sel2 = lanes == i2
    cnt_tok = sel1.astype(jnp.float32) + sel2.astype(jnp.float32)

    @pl.when(p == 0)
    def _count_pass():
        cnt_ref[pl.ds(m, 1), :] = jnp.sum(cnt_tok, axis=0, keepdims=True)

    @pl.when(p == 1)
    def _emit_pass():
        rows8 = lax.broadcasted_iota(jnp.int32, (M_T, E), 0)
        cnt_all = cnt_ref[...]
        running = jnp.sum(jnp.where(rows8 < m, cnt_all, 0.0), axis=0,
                          keepdims=True)
        totals = jnp.sum(cnt_all, axis=0, keepdims=True)
        padded = 256.0 * jnp.floor((totals + 255.0) / 256.0)
        le = lax.broadcasted_iota(jnp.int32, (E, E), 0)
        ri = lax.broadcasted_iota(jnp.int32, (E, E), 1)
        triu_s = (le < ri).astype(jnp.float32)
        base = jnp.round(jnp.dot(padded, triu_s,
                                 preferred_element_type=jnp.float32))

        r2 = lax.broadcasted_iota(jnp.int32, (TILE_M, TILE_M), 0)
        c2 = lax.broadcasted_iota(jnp.int32, (TILE_M, TILE_M), 1)
        tril_s = (c2 < r2).astype(jnp.float32)
        within = jnp.round(jnp.dot(tril_s, cnt_tok,
                                   preferred_element_type=jnp.float32))
        cpos = base + running + within
        pos0 = jnp.sum(jnp.where(sel1, cpos, 0.0), axis=-1, keepdims=True)
        pos1 = jnp.sum(jnp.where(sel2, cpos, 0.0), axis=-1, keepdims=True)
        pos0_ref[...] = pos0.astype(jnp.int32).T.reshape(1, 1, TILE_M)
        pos1_ref[...] = pos1.astype(jnp.int32).T.reshape(1, 1, TILE_M)

        w1 = jnp.sum(jnp.where(sel1, prob, 0.0), axis=-1, keepdims=True)
        w2 = jnp.sum(jnp.where(sel2, prob, 0.0), axis=-1, keepdims=True)
        sig = jax.nn.sigmoid(
            jnp.dot(xf, sg_ref[...].T, preferred_element_type=jnp.float32))
        wcomb_ref[...] = jnp.concatenate(
            [w1, w2, sig, jnp.zeros_like(w1)], axis=1)

        @pl.when(m == M_T - 1)
        def _tiles():
            ti = lax.broadcasted_iota(jnp.int32, (1, NW), 1).astype(
                jnp.float32)
            end_t = (base + padded) / 256.0
            nfin = jnp.zeros((1, NW), jnp.float32)
            for e in range(E):
                nfin = nfin + (ti >= end_t[0, e]).astype(jnp.float32)
            te = jnp.where(nfin < float(E), nfin, -1.0)
            tile_e_ref[...] = te.astype(jnp.int32)


def _run_router(x32, gate_w, shared_gate_w):
    return pl.pallas_call(
        _router_kernel,
        grid=(2, M_T),
        in_specs=[
            pl.BlockSpec((TILE_M, H), lambda p, m: (m, 0)),
            pl.BlockSpec((E, H), lambda p, m: (0, 0)),
            pl.BlockSpec((1, H), lambda p, m: (0, 0)),
        ],
        out_specs=[
            pl.BlockSpec((1, 1, TILE_M),
                         lambda p, m: (jnp.where(p == 1, m, 0), 0, 0)),
            pl.BlockSpec((1, 1, TILE_M),
                         lambda p, m: (jnp.where(p == 1, m, 0), 0, 0)),
            pl.BlockSpec((TILE_M, 4),
                         lambda p, m: (jnp.where(p == 1, m, 0), 0)),
            pl.BlockSpec((1, NW), lambda p, m: (0, 0)),
        ],
        out_shape=[
            jax.ShapeDtypeStruct((M_T, 1, TILE_M), jnp.int32),
            jax.ShapeDtypeStruct((M_T, 1, TILE_M), jnp.int32),
            jax.ShapeDtypeStruct((T, 4), jnp.float32),
            jax.ShapeDtypeStruct((1, NW), jnp.int32),
        ],
        scratch_shapes=[pltpu.VMEM((M_T, E), jnp.float32)],
        compiler_params=pltpu.CompilerParams(
            dimension_semantics=("arbitrary", "arbitrary")),
    )(x32, gate_w, shared_gate_w)


def _make_sc_scatter():
    mesh = plsc.VectorSubcoreMesh(core_axis_name="c", subcore_axis_name="s")

    @functools.partial(
        pl.kernel, mesh=mesh,
        out_type=jax.ShapeDtypeStruct((P_MAX, H), jnp.float32),
        scratch_types=[
            pltpu.VMEM((TPW, H), jnp.float32),
            pltpu.VMEM((TPW,), jnp.int32),
            pltpu.VMEM((TPW,), jnp.int32),
            pltpu.SemaphoreType.DMA,
            pltpu.SemaphoreType.DMA,
        ],
    )
    def sc_scatter(xb_hbm, pos0_hbm, pos1_hbm, xg_hbm, xloc, idx0, idx1,
                   sem0, sem1):
        wid = lax.axis_index("s") * 2 + lax.axis_index("c")
        pltpu.sync_copy(xb_hbm.at[pl.ds(wid * TPW, TPW)], xloc)
        pltpu.sync_copy(pos0_hbm.at[wid, 0], idx0)
        pltpu.sync_copy(pos1_hbm.at[wid, 0], idx1)
        c0 = pltpu.async_copy(xloc, xg_hbm.at[idx0], sem0)
        c1 = pltpu.async_copy(xloc, xg_hbm.at[idx1], sem1)
        c0.wait()
        c1.wait()

    return sc_scatter


def _grouped_kernel(te_ref, xg_ref, wg_ref, wu_ref, wd_ref, yg_ref,
                    wgb, wub, wdb):
    i = pl.program_id(0)
    te = te_ref[i]

    @pl.when(te >= 0)
    def _work():
        prev = te_ref[jnp.maximum(i - 1, 0)]

        @pl.when((i == 0) | (te != prev))
        def _cast():
            wgb[...] = wg_ref[0].astype(jnp.bfloat16)
            wub[...] = wu_ref[0].astype(jnp.bfloat16)
            wdb[...] = wd_ref[0].astype(jnp.bfloat16)

        xblk = xg_ref[...].astype(jnp.bfloat16)
        g = lax.dot_general(xblk, wgb[...], (((1,), (1,)), ((), ())),
                            preferred_element_type=jnp.float32)
        u = lax.dot_general(xblk, wub[...], (((1,), (1,)), ((), ())),
                            preferred_element_type=jnp.float32)
        h = (jax.nn.silu(g) * u).astype(jnp.bfloat16)
        d = lax.dot_general(h, wdb[...], (((1,), (1,)), ((), ())),
                            preferred_element_type=jnp.float32)
        yg_ref[...] = d


def _run_grouped(tile_e, xg2d, Wg, Wu, Wd):
    grid_spec = pltpu.PrefetchScalarGridSpec(
        num_scalar_prefetch=1,
        grid=(G_TILES,),
        in_specs=[
            pl.BlockSpec((TILE_M, H), lambda i, te: (i, 0)),
            pl.BlockSpec((1, FF, H), lambda i, te: (jnp.maximum(te[i], 0), 0, 0)),
            pl.BlockSpec((1, FF, H), lambda i, te: (jnp.maximum(te[i], 0), 0, 0)),
            pl.BlockSpec((1, H, FF), lambda i, te: (jnp.maximum(te[i], 0), 0, 0)),
        ],
        out_specs=pl.BlockSpec((TILE_M, H), lambda i, te: (i, 0)),
        scratch_shapes=[
            pltpu.VMEM((FF, H), jnp.bfloat16),
            pltpu.VMEM((FF, H), jnp.bfloat16),
            pltpu.VMEM((H, FF), jnp.bfloat16),
        ],
    )
    return pl.pallas_call(
        _grouped_kernel,
        grid_spec=grid_spec,
        out_shape=jax.ShapeDtypeStruct((P_MAX, H), jnp.float32),
        compiler_params=pltpu.CompilerParams(
            dimension_semantics=("arbitrary",),
            vmem_limit_bytes=110 * 1024 * 1024,
        ),
    )(tile_e, xg2d, Wg, Wu, Wd)


def _make_sc_gather():
    mesh = plsc.VectorSubcoreMesh(core_axis_name="c", subcore_axis_name="s")

    @functools.partial(
        pl.kernel, mesh=mesh,
        out_type=[
            jax.ShapeDtypeStruct((T, H), jnp.float32),
            jax.ShapeDtypeStruct((T, H), jnp.float32),
        ],
        scratch_types=[
            pltpu.VMEM((TPW, H), jnp.float32),
            pltpu.VMEM((TPW,), jnp.int32),
            pltpu.VMEM((TPW,), jnp.int32),
            pltpu.SemaphoreType.DMA,
        ],
    )
    def sc_gather(yg_hbm, pos0_hbm, pos1_hbm, y0_hbm, y1_hbm, rows,
                  idx0, idx1, sem0):
        wid = lax.axis_index("s") * 2 + lax.axis_index("c")
        pltpu.sync_copy(pos0_hbm.at[wid, 0], idx0)
        pltpu.sync_copy(pos1_hbm.at[wid, 0], idx1)
        pltpu.async_copy(yg_hbm.at[idx0], rows, sem0).wait()
        pltpu.sync_copy(rows, y0_hbm.at[pl.ds(wid * TPW, TPW)])
        pltpu.async_copy(yg_hbm.at[idx1], rows, sem0).wait()
        pltpu.sync_copy(rows, y1_hbm.at[pl.ds(wid * TPW, TPW)])

    return sc_gather


def _shared_kernel(xb_ref, swg_ref, swu_ref, swd_ref, wc_ref, sh_ref):
    xblk = xb_ref[...]
    g = lax.dot_general(xblk, swg_ref[...], (((1,), (1,)), ((), ())),
                        preferred_element_type=jnp.float32)
    u = lax.dot_general(xblk, swu_ref[...], (((1,), (1,)), ((), ())),
                        preferred_element_type=jnp.float32)
    h = (jax.nn.silu(g) * u).astype(jnp.bfloat16)
    sh = lax.dot_general(h, swd_ref[...], (((1,), (1,)), ((), ())),
                         preferred_element_type=jnp.float32)
    sh_ref[...] = wc_ref[:, 2:3] * sh


def _run_shared(xb2d, sWg_b, sWu_b, sWd_b, wcomb):
    return pl.pallas_call(
        _shared_kernel,
        grid=(M_T,),
        in_specs=[
            pl.BlockSpec((TILE_M, H), lambda m: (m, 0)),
            pl.BlockSpec((SFF, H), lambda m: (0, 0)),
            pl.BlockSpec((SFF, H), lambda m: (0, 0)),
            pl.BlockSpec((H, SFF), lambda m: (0, 0)),
            pl.BlockSpec((TILE_M, 4), lambda m: (m, 0)),
        ],
        out_specs=pl.BlockSpec((TILE_M, H), lambda m: (m, 0)),
        out_shape=jax.ShapeDtypeStruct((T, H), jnp.float32),
        compiler_params=pltpu.CompilerParams(
            dimension_semantics=("arbitrary",),
            vmem_limit_bytes=110 * 1024 * 1024,
        ),
    )(xb2d, sWg_b, sWu_b, sWd_b, wcomb)


def _combine_kernel(sh_ref, y0_ref, y1_ref, wc_ref, out_ref):
    wc = wc_ref[...]
    out_ref[...] = (wc[:, 0:1] * y0_ref[...] + wc[:, 1:2] * y1_ref[...]
                    + sh_ref[...])


def _run_combine(sh, y0g, y1g, wcomb):
    return pl.pallas_call(
        _combine_kernel,
        grid=(M_T,),
        in_specs=[
            pl.BlockSpec((TILE_M, H), lambda m: (m, 0)),
            pl.BlockSpec((TILE_M, H), lambda m: (m, 0)),
            pl.BlockSpec((TILE_M, H), lambda m: (m, 0)),
            pl.BlockSpec((TILE_M, 4), lambda m: (m, 0)),
        ],
        out_specs=pl.BlockSpec((TILE_M, H), lambda m: (m, 0)),
        out_shape=jax.ShapeDtypeStruct((T, H), jnp.float32),
        compiler_params=pltpu.CompilerParams(
            dimension_semantics=("arbitrary",)),
    )(sh, y0g, y1g, wcomb)


_SC_CACHE = {}


def _sc_scatter():
    if "scatter" not in _SC_CACHE:
        _SC_CACHE["scatter"] = _make_sc_scatter()
    return _SC_CACHE["scatter"]


def _sc_gather():
    if "gather" not in _SC_CACHE:
        _SC_CACHE["gather"] = _make_sc_gather()
    return _SC_CACHE["gather"]


@jax.jit
def kernel(hidden_states, gate_w, Wg, Wu, Wd, sWg, sWu, sWd, shared_gate_w):
    b, s_len, h = hidden_states.shape
    x32 = hidden_states.reshape(T, H)
    xb = x32.astype(jnp.bfloat16)

    pos0, pos1, wcomb, tile_e2 = _run_router(x32, gate_w, shared_gate_w)
    p0w = pos0.reshape(NW, 1, TPW)
    p1w = pos1.reshape(NW, 1, TPW)

    xg = _sc_scatter()(x32, p0w, p1w)
    sh = _run_shared(
        xb,
        sWg.astype(jnp.bfloat16),
        sWu.astype(jnp.bfloat16),
        sWd.astype(jnp.bfloat16),
        wcomb)
    yg = _run_grouped(tile_e2.reshape(NW), xg, Wg, Wu, Wd)
    y0g, y1g = _sc_gather()(yg, p0w, p1w)
    out = _run_combine(sh, y0g, y1g, wcomb)
    return out.reshape(b, s_len, h)


# speedup vs baseline: 1.1121x; 1.1121x over previous
import functools

import jax
import jax.numpy as jnp
from jax import lax
from jax.experimental import pallas as pl
from jax.experimental.pallas import tpu as pltpu
from jax.experimental.pallas import tpu_sc as plsc

E = 8
H = 1024
FF = 1408
SFF = 2816
T = 2048
TILE_M = 256
M_T = T // TILE_M
G_TILES = 24
P_MAX = G_TILES * TILE_M
NW = 32
TPW = T // NW


def _pack_bf16(x_bf16):
    n = x_bf16.shape[1] // 2
    lo = lax.bitcast_convert_type(x_bf16[:, :n], jnp.uint16).astype(jnp.int32)
    hi = lax.bitcast_convert_type(x_bf16[:, n:], jnp.uint16).astype(jnp.int32)
    return lo | lax.shift_left(hi, 16)


def _unpack_bf16(p_i32):
    lo = lax.bitcast_convert_type(
        (p_i32 & 0xFFFF).astype(jnp.uint16), jnp.bfloat16)
    hi = lax.bitcast_convert_type(
        lax.shift_right_logical(p_i32, 16).astype(jnp.uint16), jnp.bfloat16)
    return jnp.concatenate([lo, hi], axis=1)


def _router_kernel(x_ref, gw_ref, sg_ref,
                   pos0_ref, pos1_ref, wcomb_ref, tile_e_ref, xpk_ref,
                   cnt_ref):
    p = pl.program_id(0)
    m = pl.program_id(1)

    xf = x_ref[...]

    @pl.when(p == 0)
    def _pack_x():
        xpk_ref[...] = _pack_bf16(xf.astype(jnp.bfloat16))
    logits = jnp.dot(xf, gw_ref[...].T, preferred_element_type=jnp.float32)
    prob = jax.nn.softmax(logits, axis=-1)
    lanes = lax.broadcasted_iota(jnp.int32, (TILE_M, E), 1)
    m1 = jnp.max(prob, axis=-1, keepdims=True)
    i1 = jnp.min(jnp.where(prob == m1, lanes, E), axis=-1, keepdims=True)
    sel1 = lanes == i1
    pm = jnp.where(sel1, -jnp.inf, prob)
    m2 = jnp.max(pm, axis=-1, keepdims=True)
    i2 = jnp.min(jnp.where(pm == m2, lanes, E), axis=-1, keepdims=True)
    sel2 = lanes == i2
    cnt_tok = sel1.astype(jnp.float32) + sel2.astype(jnp.float32)

    @pl.when(p == 0)
    def _count_pass():
        cnt_ref[pl.ds(m, 1), :] = jnp.sum(cnt_tok, axis=0, keepdims=True)

    @pl.when(p == 1)
    def _emit_pass():
        rows8 = lax.broadcasted_iota(jnp.int32, (M_T, E), 0)
        cnt_all = cnt_ref[...]
        running = jnp.sum(jnp.where(rows8 < m, cnt_all, 0.0), axis=0,
                          keepdims=True)
        totals = jnp.sum(cnt_all, axis=0, keepdims=True)
        padded = 256.0 * jnp.floor((totals + 255.0) / 256.0)
        le = lax.broadcasted_iota(jnp.int32, (E, E), 0)
        ri = lax.broadcasted_iota(jnp.int32, (E, E), 1)
        triu_s = (le < ri).astype(jnp.float32)
        base = jnp.round(jnp.dot(padded, triu_s,
                                 preferred_element_type=jnp.float32))

        r2 = lax.broadcasted_iota(jnp.int32, (TILE_M, TILE_M), 0)
        c2 = lax.broadcasted_iota(jnp.int32, (TILE_M, TILE_M), 1)
        tril_s = (c2 < r2).astype(jnp.float32)
        within = jnp.round(jnp.dot(tril_s, cnt_tok,
                                   preferred_element_type=jnp.float32))
        cpos = base + running + within
        pos0 = jnp.sum(jnp.where(sel1, cpos, 0.0), axis=-1, keepdims=True)
        pos1 = jnp.sum(jnp.where(sel2, cpos, 0.0), axis=-1, keepdims=True)
        pos0_ref[...] = pos0.astype(jnp.int32).T.reshape(1, 1, TILE_M)
        pos1_ref[...] = pos1.astype(jnp.int32).T.reshape(1, 1, TILE_M)

        w1 = jnp.sum(jnp.where(sel1, prob, 0.0), axis=-1, keepdims=True)
        w2 = jnp.sum(jnp.where(sel2, prob, 0.0), axis=-1, keepdims=True)
        sig = jax.nn.sigmoid(
            jnp.dot(xf, sg_ref[...].T, preferred_element_type=jnp.float32))
        wcomb_ref[...] = jnp.concatenate(
            [w1, w2, sig, jnp.zeros_like(w1)], axis=1)

        @pl.when(m == M_T - 1)
        def _tiles():
            ti = lax.broadcasted_iota(jnp.int32, (1, NW), 1).astype(
                jnp.float32)
            end_t = (base + padded) / 256.0
            nfin = jnp.zeros((1, NW), jnp.float32)
            for e in range(E):
                nfin = nfin + (ti >= end_t[0, e]).astype(jnp.float32)
            te = jnp.where(nfin < float(E), nfin, -1.0)
            tile_e_ref[...] = te.astype(jnp.int32)


def _run_router(x32, gate_w, shared_gate_w):
    return pl.pallas_call(
        _router_kernel,
        grid=(2, M_T),
        in_specs=[
            pl.BlockSpec((TILE_M, H), lambda p, m: (m, 0)),
            pl.BlockSpec((E, H), lambda p, m: (0, 0)),
            pl.BlockSpec((1, H), lambda p, m: (0, 0)),
        ],
        out_specs=[
            pl.BlockSpec((1, 1, TILE_M),
                         lambda p, m: (jnp.where(p == 1, m, 0), 0, 0)),
            pl.BlockSpec((1, 1, TILE_M),
                         lambda p, m: (jnp.where(p == 1, m, 0), 0, 0)),
            pl.BlockSpec((TILE_M, 4),
                         lambda p, m: (jnp.where(p == 1, m, 0), 0)),
            pl.BlockSpec((1, NW), lambda p, m: (0, 0)),
            pl.BlockSpec((TILE_M, H // 2),
                         lambda p, m: (jnp.where(p == 0, m, M_T - 1), 0)),
        ],
        out_shape=[
            jax.ShapeDtypeStruct((M_T, 1, TILE_M), jnp.int32),
            jax.ShapeDtypeStruct((M_T, 1, TILE_M), jnp.int32),
            jax.ShapeDtypeStruct((T, 4), jnp.float32),
            jax.ShapeDtypeStruct((1, NW), jnp.int32),
            jax.ShapeDtypeStruct((T, H // 2), jnp.int32),
        ],
        scratch_shapes=[pltpu.VMEM((M_T, E), jnp.float32)],
        compiler_params=pltpu.CompilerParams(
            dimension_semantics=("arbitrary", "arbitrary")),
    )(x32, gate_w, shared_gate_w)


def _make_sc_scatter():
    mesh = plsc.VectorSubcoreMesh(core_axis_name="c", subcore_axis_name="s")

    @functools.partial(
        pl.kernel, mesh=mesh,
        out_type=jax.ShapeDtypeStruct((P_MAX, H // 2), jnp.int32),
        scratch_types=[
            pltpu.VMEM((TPW, H // 2), jnp.int32),
            pltpu.VMEM((TPW,), jnp.int32),
            pltpu.VMEM((TPW,), jnp.int32),
            pltpu.SemaphoreType.DMA,
            pltpu.SemaphoreType.DMA,
        ],
    )
    def sc_scatter(xb_hbm, pos0_hbm, pos1_hbm, xg_hbm, xloc, idx0, idx1,
                   sem0, sem1):
        wid = lax.axis_index("s") * 2 + lax.axis_index("c")
        pltpu.sync_copy(xb_hbm.at[pl.ds(wid * TPW, TPW)], xloc)
        pltpu.sync_copy(pos0_hbm.at[wid, 0], idx0)
        pltpu.sync_copy(pos1_hbm.at[wid, 0], idx1)
        c0 = pltpu.async_copy(xloc, xg_hbm.at[idx0], sem0)
        c1 = pltpu.async_copy(xloc, xg_hbm.at[idx1], sem1)
        c0.wait()
        c1.wait()

    return sc_scatter


def _grouped_kernel(te_ref, xg_ref, wg_ref, wu_ref, wd_ref, yg_ref,
                    wgb, wub, wdb):
    i = pl.program_id(0)
    te = te_ref[i]

    @pl.when(te >= 0)
    def _work():
        prev = te_ref[jnp.maximum(i - 1, 0)]

        @pl.when((i == 0) | (te != prev))
        def _cast():
            wgb[...] = wg_ref[0].astype(jnp.bfloat16)
            wub[...] = wu_ref[0].astype(jnp.bfloat16)
            wdb[...] = wd_ref[0].astype(jnp.bfloat16)

        xblk = _unpack_bf16(xg_ref[...])
        g = lax.dot_general(xblk, wgb[...], (((1,), (1,)), ((), ())),
                            preferred_element_type=jnp.float32)
        u = lax.dot_general(xblk, wub[...], (((1,), (1,)), ((), ())),
                            preferred_element_type=jnp.float32)
        h = (jax.nn.silu(g) * u).astype(jnp.bfloat16)
        d = lax.dot_general(h, wdb[...], (((1,), (1,)), ((), ())),
                            preferred_element_type=jnp.float32)
        yg_ref[...] = _pack_bf16(d.astype(jnp.bfloat16))


def _run_grouped(tile_e, xg2d, Wg, Wu, Wd):
    grid_spec = pltpu.PrefetchScalarGridSpec(
        num_scalar_prefetch=1,
        grid=(G_TILES,),
        in_specs=[
            pl.BlockSpec((TILE_M, H // 2), lambda i, te: (i, 0)),
            pl.BlockSpec((1, FF, H), lambda i, te: (jnp.maximum(te[i], 0), 0, 0)),
            pl.BlockSpec((1, FF, H), lambda i, te: (jnp.maximum(te[i], 0), 0, 0)),
            pl.BlockSpec((1, H, FF), lambda i, te: (jnp.maximum(te[i], 0), 0, 0)),
        ],
        out_specs=pl.BlockSpec((TILE_M, H // 2), lambda i, te: (i, 0)),
        scratch_shapes=[
            pltpu.VMEM((FF, H), jnp.bfloat16),
            pltpu.VMEM((FF, H), jnp.bfloat16),
            pltpu.VMEM((H, FF), jnp.bfloat16),
        ],
    )
    return pl.pallas_call(
        _grouped_kernel,
        grid_spec=grid_spec,
        out_shape=jax.ShapeDtypeStruct((P_MAX, H // 2), jnp.int32),
        compiler_params=pltpu.CompilerParams(
            dimension_semantics=("arbitrary",),
            vmem_limit_bytes=110 * 1024 * 1024,
        ),
    )(tile_e, xg2d, Wg, Wu, Wd)


def _make_sc_gather():
    mesh = plsc.VectorSubcoreMesh(core_axis_name="c", subcore_axis_name="s")

    @functools.partial(
        pl.kernel, mesh=mesh,
        out_type=[
            jax.ShapeDtypeStruct((T, H // 2), jnp.int32),
            jax.ShapeDtypeStruct((T, H // 2), jnp.int32),
        ],
        scratch_types=[
            pltpu.VMEM((TPW, H // 2), jnp.int32),
            pltpu.VMEM((TPW, H // 2), jnp.int32),
            pltpu.VMEM((TPW,), jnp.int32),
            pltpu.VMEM((TPW,), jnp.int32),
            pltpu.SemaphoreType.DMA,
            pltpu.SemaphoreType.DMA,
        ],
    )
    def sc_gather(yg_hbm, pos0_hbm, pos1_hbm, y0_hbm, y1_hbm, rows0, rows1,
                  idx0, idx1, sem0, sem1):
        wid = lax.axis_index("s") * 2 + lax.axis_index("c")
        pltpu.sync_copy(pos0_hbm.at[wid, 0], idx0)
        pltpu.sync_copy(pos1_hbm.at[wid, 0], idx1)
        c0 = pltpu.async_copy(yg_hbm.at[idx0], rows0, sem0)
        c1 = pltpu.async_copy(yg_hbm.at[idx1], rows1, sem1)
        c0.wait()
        c1.wait()
        pltpu.sync_copy(rows0, y0_hbm.at[pl.ds(wid * TPW, TPW)])
        pltpu.sync_copy(rows1, y1_hbm.at[pl.ds(wid * TPW, TPW)])

    return sc_gather


def _combine_kernel(xb_ref, swg_ref, swu_ref, swd_ref, y0_ref, y1_ref,
                    wc_ref, out_ref):
    xblk = xb_ref[...]
    g = lax.dot_general(xblk, swg_ref[...], (((1,), (1,)), ((), ())),
                        preferred_element_type=jnp.float32)
    u = lax.dot_general(xblk, swu_ref[...], (((1,), (1,)), ((), ())),
                        preferred_element_type=jnp.float32)
    h = (jax.nn.silu(g) * u).astype(jnp.bfloat16)
    sh = lax.dot_general(h, swd_ref[...], (((1,), (1,)), ((), ())),
                         preferred_element_type=jnp.float32)
    wc = wc_ref[...]
    y0 = _unpack_bf16(y0_ref[...]).astype(jnp.float32)
    y1 = _unpack_bf16(y1_ref[...]).astype(jnp.float32)
    out_ref[...] = wc[:, 0:1] * y0 + wc[:, 1:2] * y1 + wc[:, 2:3] * sh


def _run_combine(xb2d, sWg_b, sWu_b, sWd_b, y0g, y1g, wcomb):
    return pl.pallas_call(
        _combine_kernel,
        grid=(M_T,),
        in_specs=[
            pl.BlockSpec((TILE_M, H), lambda m: (m, 0)),
            pl.BlockSpec((SFF, H), lambda m: (0, 0)),
            pl.BlockSpec((SFF, H), lambda m: (0, 0)),
            pl.BlockSpec((H, SFF), lambda m: (0, 0)),
            pl.BlockSpec((TILE_M, H // 2), lambda m: (m, 0)),
            pl.BlockSpec((TILE_M, H // 2), lambda m: (m, 0)),
            pl.BlockSpec((TILE_M, 4), lambda m: (m, 0)),
        ],
        out_specs=pl.BlockSpec((TILE_M, H), lambda m: (m, 0)),
        out_shape=jax.ShapeDtypeStruct((T, H), jnp.float32),
        compiler_params=pltpu.CompilerParams(
            dimension_semantics=("arbitrary",),
            vmem_limit_bytes=110 * 1024 * 1024,
        ),
    )(xb2d, sWg_b, sWu_b, sWd_b, y0g, y1g, wcomb)


_SC_CACHE = {}


def _sc_scatter():
    if "scatter" not in _SC_CACHE:
        _SC_CACHE["scatter"] = _make_sc_scatter()
    return _SC_CACHE["scatter"]


def _sc_gather():
    if "gather" not in _SC_CACHE:
        _SC_CACHE["gather"] = _make_sc_gather()
    return _SC_CACHE["gather"]


@jax.jit
def kernel(hidden_states, gate_w, Wg, Wu, Wd, sWg, sWu, sWd, shared_gate_w):
    b, s_len, h = hidden_states.shape
    x32 = hidden_states.reshape(T, H)
    xb = x32.astype(jnp.bfloat16)

    pos0, pos1, wcomb, tile_e2, xpk = _run_router(x32, gate_w, shared_gate_w)
    p0w = pos0.reshape(NW, 1, TPW)
    p1w = pos1.reshape(NW, 1, TPW)

    xg = _sc_scatter()(xpk, p0w, p1w)
    yg = _run_grouped(tile_e2.reshape(NW), xg, Wg, Wu, Wd)
    y0g, y1g = _sc_gather()(yg, p0w, p1w)
    out = _run_combine(
        xb,
        sWg.astype(jnp.bfloat16),
        sWu.astype(jnp.bfloat16),
        sWd.astype(jnp.bfloat16),
        y0g, y1g, wcomb)
    return out.reshape(b, s_len, h)


# speedup vs baseline: 1.1824x; 1.0632x over previous
import functools

import jax
import jax.numpy as jnp
from jax import lax
from jax.experimental import pallas as pl
from jax.experimental.pallas import tpu as pltpu
from jax.experimental.pallas import tpu_sc as plsc

E = 8
H = 1024
FF = 1408
SFF = 2816
T = 2048
TILE_M = 256
M_T = T // TILE_M
G_TILES = 24
P_MAX = G_TILES * TILE_M
NW = 32
TPW = T // NW


def _pack_bf16(x_bf16):
    n = x_bf16.shape[1] // 2
    lo = lax.bitcast_convert_type(x_bf16[:, :n], jnp.uint16).astype(jnp.int32)
    hi = lax.bitcast_convert_type(x_bf16[:, n:], jnp.uint16).astype(jnp.int32)
    return lo | lax.shift_left(hi, 16)


def _unpack_bf16(p_i32):
    lo = lax.bitcast_convert_type(
        (p_i32 & 0xFFFF).astype(jnp.uint16), jnp.bfloat16)
    hi = lax.bitcast_convert_type(
        lax.shift_right_logical(p_i32, 16).astype(jnp.uint16), jnp.bfloat16)
    return jnp.concatenate([lo, hi], axis=1)


def _router_kernel(x_ref, gw_ref, sg_ref,
                   pos0_ref, pos1_ref, wcomb_ref, tile_e_ref, xpk_ref,
                   cnt_ref):
    p = pl.program_id(0)
    m = pl.program_id(1)

    xf = x_ref[...]

    @pl.when(p == 0)
    def _pack_x():
        xpk_ref[...] = _pack_bf16(xf.astype(jnp.bfloat16))
    logits = jnp.dot(xf, gw_ref[...].T, preferred_element_type=jnp.float32)
    prob = jax.nn.softmax(logits, axis=-1)
    lanes = lax.broadcasted_iota(jnp.int32, (TILE_M, E), 1)
    m1 = jnp.max(prob, axis=-1, keepdims=True)
    i1 = jnp.min(jnp.where(prob == m1, lanes, E), axis=-1, keepdims=True)
    sel1 = lanes == i1
    pm = jnp.where(sel1, -jnp.inf, prob)
    m2 = jnp.max(pm, axis=-1, keepdims=True)
    i2 = jnp.min(jnp.where(pm == m2, lanes, E), axis=-1, keepdims=True)
    sel2 = lanes == i2
    cnt_tok = sel1.astype(jnp.float32) + sel2.astype(jnp.float32)

    @pl.when(p == 0)
    def _count_pass():
        cnt_ref[pl.ds(m, 1), :] = jnp.sum(cnt_tok, axis=0, keepdims=True)

    @pl.when(p == 1)
    def _emit_pass():
        rows8 = lax.broadcasted_iota(jnp.int32, (M_T, E), 0)
        cnt_all = cnt_ref[...]
        running = jnp.sum(jnp.where(rows8 < m, cnt_all, 0.0), axis=0,
                          keepdims=True)
        totals = jnp.sum(cnt_all, axis=0, keepdims=True)
        padded = 256.0 * jnp.floor((totals + 255.0) / 256.0)
        le = lax.broadcasted_iota(jnp.int32, (E, E), 0)
        ri = lax.broadcasted_iota(jnp.int32, (E, E), 1)
        triu_s = (le < ri).astype(jnp.float32)
        base = jnp.round(jnp.dot(padded, triu_s,
                                 preferred_element_type=jnp.float32))

        r2 = lax.broadcasted_iota(jnp.int32, (TILE_M, TILE_M), 0)
        c2 = lax.broadcasted_iota(jnp.int32, (TILE_M, TILE_M), 1)
        tril_s = (c2 < r2).astype(jnp.float32)
        within = jnp.round(jnp.dot(tril_s, cnt_tok,
                                   preferred_element_type=jnp.float32))
        cpos = base + running + within
        pos0 = jnp.sum(jnp.where(sel1, cpos, 0.0), axis=-1, keepdims=True)
        pos1 = jnp.sum(jnp.where(sel2, cpos, 0.0), axis=-1, keepdims=True)
        pos0_ref[...] = pos0.astype(jnp.int32).T.reshape(1, 1, TILE_M)
        pos1_ref[...] = pos1.astype(jnp.int32).T.reshape(1, 1, TILE_M)

        w1 = jnp.sum(jnp.where(sel1, prob, 0.0), axis=-1, keepdims=True)
        w2 = jnp.sum(jnp.where(sel2, prob, 0.0), axis=-1, keepdims=True)
        sig = jax.nn.sigmoid(
            jnp.dot(xf, sg_ref[...].T, preferred_element_type=jnp.float32))
        wcomb_ref[...] = jnp.concatenate(
            [w1, w2, sig, jnp.zeros_like(w1)], axis=1)

        @pl.when(m == M_T - 1)
        def _tiles():
            ti = lax.broadcasted_iota(jnp.int32, (1, NW), 1).astype(
                jnp.float32)
            end_t = (base + padded) / 256.0
            nfin = jnp.zeros((1, NW), jnp.float32)
            for e in range(E):
                nfin = nfin + (ti >= end_t[0, e]).astype(jnp.float32)
            eids = lax.broadcasted_iota(jnp.int32, (1, E), 1).astype(
                jnp.float32)
            e_last = jnp.max(jnp.where(padded > 0.0, eids, -1.0))
            te = jnp.where(nfin < float(E), nfin, e_last + float(E))
            tile_e_ref[...] = te.astype(jnp.int32)


def _run_router(x32, gate_w, shared_gate_w):
    return pl.pallas_call(
        _router_kernel,
        grid=(2, M_T),
        in_specs=[
            pl.BlockSpec((TILE_M, H), lambda p, m: (m, 0)),
            pl.BlockSpec((E, H), lambda p, m: (0, 0)),
            pl.BlockSpec((1, H), lambda p, m: (0, 0)),
        ],
        out_specs=[
            pl.BlockSpec((1, 1, TILE_M),
                         lambda p, m: (jnp.where(p == 1, m, 0), 0, 0)),
            pl.BlockSpec((1, 1, TILE_M),
                         lambda p, m: (jnp.where(p == 1, m, 0), 0, 0)),
            pl.BlockSpec((TILE_M, 4),
                         lambda p, m: (jnp.where(p == 1, m, 0), 0)),
            pl.BlockSpec((1, NW), lambda p, m: (0, 0)),
            pl.BlockSpec((TILE_M, H // 2),
                         lambda p, m: (jnp.where(p == 0, m, M_T - 1), 0)),
        ],
        out_shape=[
            jax.ShapeDtypeStruct((M_T, 1, TILE_M), jnp.int32),
            jax.ShapeDtypeStruct((M_T, 1, TILE_M), jnp.int32),
            jax.ShapeDtypeStruct((T, 4), jnp.float32),
            jax.ShapeDtypeStruct((1, NW), jnp.int32),
            jax.ShapeDtypeStruct((T, H // 2), jnp.int32),
        ],
        scratch_shapes=[pltpu.VMEM((M_T, E), jnp.float32)],
        compiler_params=pltpu.CompilerParams(
            dimension_semantics=("arbitrary", "arbitrary")),
    )(x32, gate_w, shared_gate_w)


def _make_sc_scatter():
    mesh = plsc.VectorSubcoreMesh(core_axis_name="c", subcore_axis_name="s")

    @functools.partial(
        pl.kernel, mesh=mesh,
        out_type=jax.ShapeDtypeStruct((P_MAX, H // 2), jnp.int32),
        scratch_types=[
            pltpu.VMEM((TPW, H // 2), jnp.int32),
            pltpu.VMEM((TPW,), jnp.int32),
            pltpu.VMEM((TPW,), jnp.int32),
            pltpu.SemaphoreType.DMA,
            pltpu.SemaphoreType.DMA,
        ],
    )
    def sc_scatter(xb_hbm, pos0_hbm, pos1_hbm, xg_hbm, xloc, idx0, idx1,
                   sem0, sem1):
        wid = lax.axis_index("s") * 2 + lax.axis_index("c")
        pltpu.sync_copy(xb_hbm.at[pl.ds(wid * TPW, TPW)], xloc)
        pltpu.sync_copy(pos0_hbm.at[wid, 0], idx0)
        pltpu.sync_copy(pos1_hbm.at[wid, 0], idx1)
        c0 = pltpu.async_copy(xloc, xg_hbm.at[idx0], sem0)
        c1 = pltpu.async_copy(xloc, xg_hbm.at[idx1], sem1)
        c0.wait()
        c1.wait()

    return sc_scatter


def _grouped_kernel(te_ref, xg_ref, wg_ref, wu_ref, wd_ref, yg_ref,
                    wgb, wub, wdb):
    i = pl.program_id(0)
    te = te_ref[i]

    @pl.when(te < E)
    def _work():
        prev = te_ref[jnp.maximum(i - 1, 0)]

        @pl.when((i == 0) | (te != prev))
        def _cast():
            wgb[...] = wg_ref[0].astype(jnp.bfloat16)
            wub[...] = wu_ref[0].astype(jnp.bfloat16)
            wdb[...] = wd_ref[0].astype(jnp.bfloat16)

        xblk = _unpack_bf16(xg_ref[...])
        g = lax.dot_general(xblk, wgb[...], (((1,), (1,)), ((), ())),
                            preferred_element_type=jnp.float32)
        u = lax.dot_general(xblk, wub[...], (((1,), (1,)), ((), ())),
                            preferred_element_type=jnp.float32)
        h = (jax.nn.silu(g) * u).astype(jnp.bfloat16)
        d = lax.dot_general(h, wdb[...], (((1,), (1,)), ((), ())),
                            preferred_element_type=jnp.float32)
        yg_ref[...] = _pack_bf16(d.astype(jnp.bfloat16))


def _run_grouped(tile_e, xg2d, Wg, Wu, Wd):
    grid_spec = pltpu.PrefetchScalarGridSpec(
        num_scalar_prefetch=1,
        grid=(G_TILES,),
        in_specs=[
            pl.BlockSpec((TILE_M, H // 2), lambda i, te: (i, 0)),
            pl.BlockSpec((1, FF, H),
                         lambda i, te: (lax.rem(te[i], E), 0, 0)),
            pl.BlockSpec((1, FF, H),
                         lambda i, te: (lax.rem(te[i], E), 0, 0)),
            pl.BlockSpec((1, H, FF),
                         lambda i, te: (lax.rem(te[i], E), 0, 0)),
        ],
        out_specs=pl.BlockSpec((TILE_M, H // 2), lambda i, te: (i, 0)),
        scratch_shapes=[
            pltpu.VMEM((FF, H), jnp.bfloat16),
            pltpu.VMEM((FF, H), jnp.bfloat16),
            pltpu.VMEM((H, FF), jnp.bfloat16),
        ],
    )
    return pl.pallas_call(
        _grouped_kernel,
        grid_spec=grid_spec,
        out_shape=jax.ShapeDtypeStruct((P_MAX, H // 2), jnp.int32),
        compiler_params=pltpu.CompilerParams(
            dimension_semantics=("arbitrary",),
            vmem_limit_bytes=110 * 1024 * 1024,
        ),
    )(tile_e, xg2d, Wg, Wu, Wd)


def _make_sc_gather():
    mesh = plsc.VectorSubcoreMesh(core_axis_name="c", subcore_axis_name="s")

    @functools.partial(
        pl.kernel, mesh=mesh,
        out_type=[
            jax.ShapeDtypeStruct((T, H // 2), jnp.int32),
            jax.ShapeDtypeStruct((T, H // 2), jnp.int32),
        ],
        scratch_types=[
            pltpu.VMEM((TPW, H // 2), jnp.int32),
            pltpu.VMEM((TPW, H // 2), jnp.int32),
            pltpu.VMEM((TPW,), jnp.int32),
            pltpu.VMEM((TPW,), jnp.int32),
            pltpu.SemaphoreType.DMA,
            pltpu.SemaphoreType.DMA,
        ],
    )
    def sc_gather(yg_hbm, pos0_hbm, pos1_hbm, y0_hbm, y1_hbm, rows0, rows1,
                  idx0, idx1, sem0, sem1):
        wid = lax.axis_index("s") * 2 + lax.axis_index("c")
        pltpu.sync_copy(pos0_hbm.at[wid, 0], idx0)
        pltpu.sync_copy(pos1_hbm.at[wid, 0], idx1)
        c0 = pltpu.async_copy(yg_hbm.at[idx0], rows0, sem0)
        c1 = pltpu.async_copy(yg_hbm.at[idx1], rows1, sem1)
        c0.wait()
        c1.wait()
        pltpu.sync_copy(rows0, y0_hbm.at[pl.ds(wid * TPW, TPW)])
        pltpu.sync_copy(rows1, y1_hbm.at[pl.ds(wid * TPW, TPW)])

    return sc_gather


def _combine_kernel(xb_ref, swg_ref, swu_ref, swd_ref, y0_ref, y1_ref,
                    wc_ref, out_ref, swg_s, swu_s, swd_s):
    m = pl.program_id(0)

    @pl.when(m == 0)
    def _cast():
        swg_s[...] = swg_ref[...].astype(jnp.bfloat16)
        swu_s[...] = swu_ref[...].astype(jnp.bfloat16)
        swd_s[...] = swd_ref[...].astype(jnp.bfloat16)

    xblk = xb_ref[...]
    g = lax.dot_general(xblk, swg_s[...], (((1,), (1,)), ((), ())),
                        preferred_element_type=jnp.float32)
    u = lax.dot_general(xblk, swu_s[...], (((1,), (1,)), ((), ())),
                        preferred_element_type=jnp.float32)
    h = (jax.nn.silu(g) * u).astype(jnp.bfloat16)
    sh = lax.dot_general(h, swd_s[...], (((1,), (1,)), ((), ())),
                         preferred_element_type=jnp.float32)
    wc = wc_ref[...]
    y0 = _unpack_bf16(y0_ref[...]).astype(jnp.float32)
    y1 = _unpack_bf16(y1_ref[...]).astype(jnp.float32)
    out_ref[...] = wc[:, 0:1] * y0 + wc[:, 1:2] * y1 + wc[:, 2:3] * sh


def _run_combine(xb2d, sWg_f, sWu_f, sWd_f, y0g, y1g, wcomb):
    return pl.pallas_call(
        _combine_kernel,
        grid=(M_T,),
        in_specs=[
            pl.BlockSpec((TILE_M, H), lambda m: (m, 0)),
            pl.BlockSpec((SFF, H), lambda m: (0, 0)),
            pl.BlockSpec((SFF, H), lambda m: (0, 0)),
            pl.BlockSpec((H, SFF), lambda m: (0, 0)),
            pl.BlockSpec((TILE_M, H // 2), lambda m: (m, 0)),
            pl.BlockSpec((TILE_M, H // 2), lambda m: (m, 0)),
            pl.BlockSpec((TILE_M, 4), lambda m: (m, 0)),
        ],
        out_specs=pl.BlockSpec((TILE_M, H), lambda m: (m, 0)),
        out_shape=jax.ShapeDtypeStruct((T, H), jnp.float32),
        scratch_shapes=[
            pltpu.VMEM((SFF, H), jnp.bfloat16),
            pltpu.VMEM((SFF, H), jnp.bfloat16),
            pltpu.VMEM((H, SFF), jnp.bfloat16),
        ],
        compiler_params=pltpu.CompilerParams(
            dimension_semantics=("arbitrary",),
            vmem_limit_bytes=110 * 1024 * 1024,
        ),
    )(xb2d, sWg_f, sWu_f, sWd_f, y0g, y1g, wcomb)


_SC_CACHE = {}


def _sc_scatter():
    if "scatter" not in _SC_CACHE:
        _SC_CACHE["scatter"] = _make_sc_scatter()
    return _SC_CACHE["scatter"]


def _sc_gather():
    if "gather" not in _SC_CACHE:
        _SC_CACHE["gather"] = _make_sc_gather()
    return _SC_CACHE["gather"]


@jax.jit
def kernel(hidden_states, gate_w, Wg, Wu, Wd, sWg, sWu, sWd, shared_gate_w):
    b, s_len, h = hidden_states.shape
    x32 = hidden_states.reshape(T, H)
    xb = x32.astype(jnp.bfloat16)

    pos0, pos1, wcomb, tile_e2, xpk = _run_router(x32, gate_w, shared_gate_w)
    p0w = pos0.reshape(NW, 1, TPW)
    p1w = pos1.reshape(NW, 1, TPW)

    xg = _sc_scatter()(xpk, p0w, p1w)
    yg = _run_grouped(tile_e2.reshape(NW), xg, Wg, Wu, Wd)
    y0g, y1g = _sc_gather()(yg, p0w, p1w)
    out = _run_combine(xb, sWg, sWu, sWd, y0g, y1g, wcomb)
    return out.reshape(b, s_len, h)


# speedup vs baseline: 1.2281x; 1.0386x over previous
import functools

import jax
import jax.numpy as jnp
from jax import lax
from jax.experimental import pallas as pl
from jax.experimental.pallas import tpu as pltpu
from jax.experimental.pallas import tpu_sc as plsc

E = 8
H = 1024
FF = 1408
SFF = 2816
T = 2048
TILE_M = 256
M_T = T // TILE_M
G_TILES = 24
P_MAX = G_TILES * TILE_M
NW = 32
TPW = T // NW


def _pack_bf16(x_bf16):
    n = x_bf16.shape[1] // 2
    lo = lax.bitcast_convert_type(x_bf16[:, :n], jnp.uint16).astype(jnp.int32)
    hi = lax.bitcast_convert_type(x_bf16[:, n:], jnp.uint16).astype(jnp.int32)
    return lo | lax.shift_left(hi, 16)


def _unpack_bf16(p_i32):
    lo = lax.bitcast_convert_type(
        (p_i32 & 0xFFFF).astype(jnp.uint16), jnp.bfloat16)
    hi = lax.bitcast_convert_type(
        lax.shift_right_logical(p_i32, 16).astype(jnp.uint16), jnp.bfloat16)
    return jnp.concatenate([lo, hi], axis=1)


def _router_kernel(x_ref, gw_ref, sg_ref,
                   pos0_ref, pos1_ref, wcomb_ref, tile_e_ref, xpk_ref,
                   cnt_ref, sel1_ref, sel2_ref):
    p = pl.program_id(0)
    m = pl.program_id(1)
    msl = pl.ds(m * TILE_M, TILE_M)

    @pl.when(p == 0)
    def _count_pass():
        xf = x_ref[...]
        xpk_ref[...] = _pack_bf16(xf.astype(jnp.bfloat16))
        logits = jnp.dot(xf, gw_ref[...].T,
                         preferred_element_type=jnp.float32)
        prob = jax.nn.softmax(logits, axis=-1)
        lanes = lax.broadcasted_iota(jnp.int32, (TILE_M, E), 1)
        m1 = jnp.max(prob, axis=-1, keepdims=True)
        i1 = jnp.min(jnp.where(prob == m1, lanes, E), axis=-1, keepdims=True)
        sel1 = lanes == i1
        pm = jnp.where(sel1, -jnp.inf, prob)
        m2 = jnp.max(pm, axis=-1, keepdims=True)
        i2 = jnp.min(jnp.where(pm == m2, lanes, E), axis=-1, keepdims=True)
        sel2 = lanes == i2
        s1f = sel1.astype(jnp.float32)
        s2f = sel2.astype(jnp.float32)
        sel1_ref[msl, :] = s1f
        sel2_ref[msl, :] = s2f
        cnt_ref[pl.ds(m, 1), :] = jnp.sum(s1f + s2f, axis=0, keepdims=True)

        w1 = jnp.sum(jnp.where(sel1, prob, 0.0), axis=-1, keepdims=True)
        w2 = jnp.sum(jnp.where(sel2, prob, 0.0), axis=-1, keepdims=True)
        sig = jax.nn.sigmoid(
            jnp.dot(xf, sg_ref[...].T, preferred_element_type=jnp.float32))
        wcomb_ref[...] = jnp.concatenate(
            [w1, w2, sig, jnp.zeros_like(w1)], axis=1)

    @pl.when(p == 1)
    def _emit_pass():
        s1f = sel1_ref[msl, :]
        s2f = sel2_ref[msl, :]
        sel1 = s1f > 0.5
        sel2 = s2f > 0.5
        cnt_tok = s1f + s2f
        rows8 = lax.broadcasted_iota(jnp.int32, (M_T, E), 0)
        cnt_all = cnt_ref[...]
        running = jnp.sum(jnp.where(rows8 < m, cnt_all, 0.0), axis=0,
                          keepdims=True)
        totals = jnp.sum(cnt_all, axis=0, keepdims=True)
        padded = 256.0 * jnp.floor((totals + 255.0) / 256.0)
        le = lax.broadcasted_iota(jnp.int32, (E, E), 0)
        ri = lax.broadcasted_iota(jnp.int32, (E, E), 1)
        triu_s = (le < ri).astype(jnp.float32)
        base = jnp.round(jnp.dot(padded, triu_s,
                                 preferred_element_type=jnp.float32))

        r2 = lax.broadcasted_iota(jnp.int32, (TILE_M, TILE_M), 0)
        c2 = lax.broadcasted_iota(jnp.int32, (TILE_M, TILE_M), 1)
        tril_s = (c2 < r2).astype(jnp.float32)
        within = jnp.round(jnp.dot(tril_s, cnt_tok,
                                   preferred_element_type=jnp.float32))
        cpos = base + running + within
        pos0 = jnp.sum(jnp.where(sel1, cpos, 0.0), axis=-1, keepdims=True)
        pos1 = jnp.sum(jnp.where(sel2, cpos, 0.0), axis=-1, keepdims=True)
        pos0_ref[...] = pos0.astype(jnp.int32).T.reshape(1, 1, TILE_M)
        pos1_ref[...] = pos1.astype(jnp.int32).T.reshape(1, 1, TILE_M)

        @pl.when(m == M_T - 1)
        def _tiles():
            ti = lax.broadcasted_iota(jnp.int32, (1, NW), 1).astype(
                jnp.float32)
            end_t = (base + padded) / 256.0
            nfin = jnp.zeros((1, NW), jnp.float32)
            for e in range(E):
                nfin = nfin + (ti >= end_t[0, e]).astype(jnp.float32)
            eids = lax.broadcasted_iota(jnp.int32, (1, E), 1).astype(
                jnp.float32)
            e_last = jnp.max(jnp.where(padded > 0.0, eids, -1.0))
            te = jnp.where(nfin < float(E), nfin, e_last + float(E))
            tile_e_ref[...] = te.astype(jnp.int32)


def _run_router(x32, gate_w, shared_gate_w):
    return pl.pallas_call(
        _router_kernel,
        grid=(2, M_T),
        in_specs=[
            pl.BlockSpec((TILE_M, H),
                         lambda p, m: (jnp.where(p == 0, m, M_T - 1), 0)),
            pl.BlockSpec((E, H), lambda p, m: (0, 0)),
            pl.BlockSpec((1, H), lambda p, m: (0, 0)),
        ],
        out_specs=[
            pl.BlockSpec((1, 1, TILE_M),
                         lambda p, m: (jnp.where(p == 1, m, 0), 0, 0)),
            pl.BlockSpec((1, 1, TILE_M),
                         lambda p, m: (jnp.where(p == 1, m, 0), 0, 0)),
            pl.BlockSpec((TILE_M, 4),
                         lambda p, m: (jnp.where(p == 0, m, M_T - 1), 0)),
            pl.BlockSpec((1, NW), lambda p, m: (0, 0)),
            pl.BlockSpec((TILE_M, H // 2),
                         lambda p, m: (jnp.where(p == 0, m, M_T - 1), 0)),
        ],
        out_shape=[
            jax.ShapeDtypeStruct((M_T, 1, TILE_M), jnp.int32),
            jax.ShapeDtypeStruct((M_T, 1, TILE_M), jnp.int32),
            jax.ShapeDtypeStruct((T, 4), jnp.float32),
            jax.ShapeDtypeStruct((1, NW), jnp.int32),
            jax.ShapeDtypeStruct((T, H // 2), jnp.int32),
        ],
        scratch_shapes=[
            pltpu.VMEM((M_T, E), jnp.float32),
            pltpu.VMEM((T, E), jnp.float32),
            pltpu.VMEM((T, E), jnp.float32),
        ],
        compiler_params=pltpu.CompilerParams(
            dimension_semantics=("arbitrary", "arbitrary")),
    )(x32, gate_w, shared_gate_w)


def _make_sc_scatter():
    mesh = plsc.VectorSubcoreMesh(core_axis_name="c", subcore_axis_name="s")

    @functools.partial(
        pl.kernel, mesh=mesh,
        out_type=jax.ShapeDtypeStruct((P_MAX, H // 2), jnp.int32),
        scratch_types=[
            pltpu.VMEM((TPW, H // 2), jnp.int32),
            pltpu.VMEM((TPW,), jnp.int32),
            pltpu.VMEM((TPW,), jnp.int32),
            pltpu.SemaphoreType.DMA,
            pltpu.SemaphoreType.DMA,
        ],
    )
    def sc_scatter(xb_hbm, pos0_hbm, pos1_hbm, xg_hbm, xloc, idx0, idx1,
                   sem0, sem1):
        wid = lax.axis_index("s") * 2 + lax.axis_index("c")
        pltpu.sync_copy(xb_hbm.at[pl.ds(wid * TPW, TPW)], xloc)
        pltpu.sync_copy(pos0_hbm.at[wid, 0], idx0)
        pltpu.sync_copy(pos1_hbm.at[wid, 0], idx1)
        c0 = pltpu.async_copy(xloc, xg_hbm.at[idx0], sem0)
        c1 = pltpu.async_copy(xloc, xg_hbm.at[idx1], sem1)
        c0.wait()
        c1.wait()

    return sc_scatter


def _grouped_kernel(te_ref, xg_ref, wg_ref, wu_ref, wd_ref, yg_ref,
                    wgb, wub, wdb):
    i = pl.program_id(0)
    te = te_ref[i]

    @pl.when(te < E)
    def _work():
        prev = te_ref[jnp.maximum(i - 1, 0)]

        @pl.when((i == 0) | (te != prev))
        def _cast():
            wgb[...] = wg_ref[0].astype(jnp.bfloat16)
            wub[...] = wu_ref[0].astype(jnp.bfloat16)
            wdb[...] = wd_ref[0].astype(jnp.bfloat16)

        xblk = _unpack_bf16(xg_ref[...])
        g = lax.dot_general(xblk, wgb[...], (((1,), (1,)), ((), ())),
                            preferred_element_type=jnp.float32)
        u = lax.dot_general(xblk, wub[...], (((1,), (1,)), ((), ())),
                            preferred_element_type=jnp.float32)
        h = (jax.nn.silu(g) * u).astype(jnp.bfloat16)
        d = lax.dot_general(h, wdb[...], (((1,), (1,)), ((), ())),
                            preferred_element_type=jnp.float32)
        yg_ref[...] = _pack_bf16(d.astype(jnp.bfloat16))


def _run_grouped(tile_e, xg2d, Wg, Wu, Wd):
    grid_spec = pltpu.PrefetchScalarGridSpec(
        num_scalar_prefetch=1,
        grid=(G_TILES,),
        in_specs=[
            pl.BlockSpec((TILE_M, H // 2), lambda i, te: (i, 0)),
            pl.BlockSpec((1, FF, H),
                         lambda i, te: (lax.rem(te[i], E), 0, 0)),
            pl.BlockSpec((1, FF, H),
                         lambda i, te: (lax.rem(te[i], E), 0, 0)),
            pl.BlockSpec((1, H, FF),
                         lambda i, te: (lax.rem(te[i], E), 0, 0)),
        ],
        out_specs=pl.BlockSpec((TILE_M, H // 2), lambda i, te: (i, 0)),
        scratch_shapes=[
            pltpu.VMEM((FF, H), jnp.bfloat16),
            pltpu.VMEM((FF, H), jnp.bfloat16),
            pltpu.VMEM((H, FF), jnp.bfloat16),
        ],
    )
    return pl.pallas_call(
        _grouped_kernel,
        grid_spec=grid_spec,
        out_shape=jax.ShapeDtypeStruct((P_MAX, H // 2), jnp.int32),
        compiler_params=pltpu.CompilerParams(
            dimension_semantics=("arbitrary",),
            vmem_limit_bytes=110 * 1024 * 1024,
        ),
    )(tile_e, xg2d, Wg, Wu, Wd)


def _make_sc_gather():
    mesh = plsc.VectorSubcoreMesh(core_axis_name="c", subcore_axis_name="s")

    @functools.partial(
        pl.kernel, mesh=mesh,
        out_type=[
            jax.ShapeDtypeStruct((T, H // 2), jnp.int32),
            jax.ShapeDtypeStruct((T, H // 2), jnp.int32),
        ],
        scratch_types=[
            pltpu.VMEM((TPW, H // 2), jnp.int32),
            pltpu.VMEM((TPW, H // 2), jnp.int32),
            pltpu.VMEM((TPW,), jnp.int32),
            pltpu.VMEM((TPW,), jnp.int32),
            pltpu.SemaphoreType.DMA,
            pltpu.SemaphoreType.DMA,
        ],
    )
    def sc_gather(yg_hbm, pos0_hbm, pos1_hbm, y0_hbm, y1_hbm, rows0, rows1,
                  idx0, idx1, sem0, sem1):
        wid = lax.axis_index("s") * 2 + lax.axis_index("c")
        pltpu.sync_copy(pos0_hbm.at[wid, 0], idx0)
        pltpu.sync_copy(pos1_hbm.at[wid, 0], idx1)
        c0 = pltpu.async_copy(yg_hbm.at[idx0], rows0, sem0)
        c1 = pltpu.async_copy(yg_hbm.at[idx1], rows1, sem1)
        c0.wait()
        c1.wait()
        pltpu.sync_copy(rows0, y0_hbm.at[pl.ds(wid * TPW, TPW)])
        pltpu.sync_copy(rows1, y1_hbm.at[pl.ds(wid * TPW, TPW)])

    return sc_gather


def _combine_kernel(xb_ref, swg_ref, swu_ref, swd_ref, y0_ref, y1_ref,
                    wc_ref, out_ref, swg_s, swu_s, swd_s):
    m = pl.program_id(0)

    @pl.when(m == 0)
    def _cast():
        swg_s[...] = swg_ref[...].astype(jnp.bfloat16)
        swu_s[...] = swu_ref[...].astype(jnp.bfloat16)
        swd_s[...] = swd_ref[...].astype(jnp.bfloat16)

    xblk = xb_ref[...]
    g = lax.dot_general(xblk, swg_s[...], (((1,), (1,)), ((), ())),
                        preferred_element_type=jnp.float32)
    u = lax.dot_general(xblk, swu_s[...], (((1,), (1,)), ((), ())),
                        preferred_element_type=jnp.float32)
    h = (jax.nn.silu(g) * u).astype(jnp.bfloat16)
    sh = lax.dot_general(h, swd_s[...], (((1,), (1,)), ((), ())),
                         preferred_element_type=jnp.float32)
    wc = wc_ref[...]
    y0 = _unpack_bf16(y0_ref[...]).astype(jnp.float32)
    y1 = _unpack_bf16(y1_ref[...]).astype(jnp.float32)
    out_ref[...] = wc[:, 0:1] * y0 + wc[:, 1:2] * y1 + wc[:, 2:3] * sh


def _run_combine(xb2d, sWg_f, sWu_f, sWd_f, y0g, y1g, wcomb):
    return pl.pallas_call(
        _combine_kernel,
        grid=(M_T,),
        in_specs=[
            pl.BlockSpec((TILE_M, H), lambda m: (m, 0)),
            pl.BlockSpec((SFF, H), lambda m: (0, 0)),
            pl.BlockSpec((SFF, H), lambda m: (0, 0)),
            pl.BlockSpec((H, SFF), lambda m: (0, 0)),
            pl.BlockSpec((TILE_M, H // 2), lambda m: (m, 0)),
            pl.BlockSpec((TILE_M, H // 2), lambda m: (m, 0)),
            pl.BlockSpec((TILE_M, 4), lambda m: (m, 0)),
        ],
        out_specs=pl.BlockSpec((TILE_M, H), lambda m: (m, 0)),
        out_shape=jax.ShapeDtypeStruct((T, H), jnp.float32),
        scratch_shapes=[
            pltpu.VMEM((SFF, H), jnp.bfloat16),
            pltpu.VMEM((SFF, H), jnp.bfloat16),
            pltpu.VMEM((H, SFF), jnp.bfloat16),
        ],
        compiler_params=pltpu.CompilerParams(
            dimension_semantics=("arbitrary",),
            vmem_limit_bytes=110 * 1024 * 1024,
        ),
    )(xb2d, sWg_f, sWu_f, sWd_f, y0g, y1g, wcomb)


_SC_CACHE = {}


def _sc_scatter():
    if "scatter" not in _SC_CACHE:
        _SC_CACHE["scatter"] = _make_sc_scatter()
    return _SC_CACHE["scatter"]


def _sc_gather():
    if "gather" not in _SC_CACHE:
        _SC_CACHE["gather"] = _make_sc_gather()
    return _SC_CACHE["gather"]


@jax.jit
def kernel(hidden_states, gate_w, Wg, Wu, Wd, sWg, sWu, sWd, shared_gate_w):
    b, s_len, h = hidden_states.shape
    x32 = hidden_states.reshape(T, H)
    xb = x32.astype(jnp.bfloat16)

    pos0, pos1, wcomb, tile_e2, xpk = _run_router(x32, gate_w, shared_gate_w)
    p0w = pos0.reshape(NW, 1, TPW)
    p1w = pos1.reshape(NW, 1, TPW)

    xg = _sc_scatter()(xpk, p0w, p1w)
    yg = _run_grouped(tile_e2.reshape(NW), xg, Wg, Wu, Wd)
    y0g, y1g = _sc_gather()(yg, p0w, p1w)
    out = _run_combine(xb, sWg, sWu, sWd, y0g, y1g, wcomb)
    return out.reshape(b, s_len, h)


# speedup vs baseline: 1.2461x; 1.0147x over previous
import functools

import jax
import jax.numpy as jnp
from jax import lax
from jax.experimental import pallas as pl
from jax.experimental.pallas import tpu as pltpu
from jax.experimental.pallas import tpu_sc as plsc

E = 8
H = 1024
FF = 1408
SFF = 2816
T = 2048
TILE_M = 256
M_T = T // TILE_M
G_TILES = 24
P_MAX = G_TILES * TILE_M
NW = 32
TPW = T // NW


def _pack_bf16(x_bf16):
    n = x_bf16.shape[1] // 2
    lo = lax.bitcast_convert_type(x_bf16[:, :n], jnp.uint16).astype(jnp.int32)
    hi = lax.bitcast_convert_type(x_bf16[:, n:], jnp.uint16).astype(jnp.int32)
    return lo | lax.shift_left(hi, 16)


def _unpack_bf16(p_i32):
    lo = lax.bitcast_convert_type(
        (p_i32 & 0xFFFF).astype(jnp.uint16), jnp.bfloat16)
    hi = lax.bitcast_convert_type(
        lax.shift_right_logical(p_i32, 16).astype(jnp.uint16), jnp.bfloat16)
    return jnp.concatenate([lo, hi], axis=1)


def _router_kernel(x_ref, gw_ref, sg_ref,
                   pos0_ref, pos1_ref, wcomb_ref, tile_e_ref, xpk_ref,
                   cnt_ref, sel1_ref, sel2_ref):
    p = pl.program_id(0)
    m = pl.program_id(1)
    msl = pl.ds(m * TILE_M, TILE_M)

    @pl.when(p == 0)
    def _count_pass():
        xf = x_ref[...]
        xpk_ref[...] = _pack_bf16(xf.astype(jnp.bfloat16))
        logits = jnp.dot(xf, gw_ref[...].T,
                         preferred_element_type=jnp.float32)
        prob = jax.nn.softmax(logits, axis=-1)
        lanes = lax.broadcasted_iota(jnp.int32, (TILE_M, E), 1)
        m1 = jnp.max(prob, axis=-1, keepdims=True)
        i1 = jnp.min(jnp.where(prob == m1, lanes, E), axis=-1, keepdims=True)
        sel1 = lanes == i1
        pm = jnp.where(sel1, -jnp.inf, prob)
        m2 = jnp.max(pm, axis=-1, keepdims=True)
        i2 = jnp.min(jnp.where(pm == m2, lanes, E), axis=-1, keepdims=True)
        sel2 = lanes == i2
        s1f = sel1.astype(jnp.float32)
        s2f = sel2.astype(jnp.float32)
        sel1_ref[msl, :] = s1f
        sel2_ref[msl, :] = s2f
        cnt_ref[pl.ds(m, 1), :] = jnp.sum(s1f + s2f, axis=0, keepdims=True)

        w1 = jnp.sum(jnp.where(sel1, prob, 0.0), axis=-1, keepdims=True)
        w2 = jnp.sum(jnp.where(sel2, prob, 0.0), axis=-1, keepdims=True)
        sig = jax.nn.sigmoid(
            jnp.dot(xf, sg_ref[...].T, preferred_element_type=jnp.float32))
        wcomb_ref[...] = jnp.concatenate(
            [w1, w2, sig, jnp.zeros_like(w1)], axis=1)

    @pl.when(p == 1)
    def _emit_pass():
        s1f = sel1_ref[msl, :]
        s2f = sel2_ref[msl, :]
        sel1 = s1f > 0.5
        sel2 = s2f > 0.5
        cnt_tok = s1f + s2f
        rows8 = lax.broadcasted_iota(jnp.int32, (M_T, E), 0)
        cnt_all = cnt_ref[...]
        running = jnp.sum(jnp.where(rows8 < m, cnt_all, 0.0), axis=0,
                          keepdims=True)
        totals = jnp.sum(cnt_all, axis=0, keepdims=True)
        padded = 256.0 * jnp.floor((totals + 255.0) / 256.0)
        le = lax.broadcasted_iota(jnp.int32, (E, E), 0)
        ri = lax.broadcasted_iota(jnp.int32, (E, E), 1)
        triu_s = (le < ri).astype(jnp.float32)
        base = jnp.round(jnp.dot(padded, triu_s,
                                 preferred_element_type=jnp.float32))

        r2 = lax.broadcasted_iota(jnp.int32, (TILE_M, TILE_M), 0)
        c2 = lax.broadcasted_iota(jnp.int32, (TILE_M, TILE_M), 1)
        tril_s = (c2 < r2).astype(jnp.float32)
        within = jnp.round(jnp.dot(tril_s, cnt_tok,
                                   preferred_element_type=jnp.float32))
        cpos = base + running + within
        pos0 = jnp.sum(jnp.where(sel1, cpos, 0.0), axis=-1, keepdims=True)
        pos1 = jnp.sum(jnp.where(sel2, cpos, 0.0), axis=-1, keepdims=True)
        pos0_ref[...] = pos0.astype(jnp.int32).T.reshape(1, 1, TILE_M)
        pos1_ref[...] = pos1.astype(jnp.int32).T.reshape(1, 1, TILE_M)

        @pl.when(m == M_T - 1)
        def _tiles():
            ti = lax.broadcasted_iota(jnp.int32, (1, NW), 1).astype(
                jnp.float32)
            end_t = (base + padded) / 256.0
            nfin = jnp.zeros((1, NW), jnp.float32)
            for e in range(E):
                nfin = nfin + (ti >= end_t[0, e]).astype(jnp.float32)
            eids = lax.broadcasted_iota(jnp.int32, (1, E), 1).astype(
                jnp.float32)
            e_last = jnp.max(jnp.where(padded > 0.0, eids, -1.0))
            te = jnp.where(nfin < float(E), nfin, e_last + float(E))
            tile_e_ref[...] = te.astype(jnp.int32)


def _run_router(x32, gate_w, shared_gate_w):
    return pl.pallas_call(
        _router_kernel,
        grid=(2, M_T),
        in_specs=[
            pl.BlockSpec((TILE_M, H),
                         lambda p, m: (jnp.where(p == 0, m, M_T - 1), 0)),
            pl.BlockSpec((E, H), lambda p, m: (0, 0)),
            pl.BlockSpec((1, H), lambda p, m: (0, 0)),
        ],
        out_specs=[
            pl.BlockSpec((1, 1, TILE_M),
                         lambda p, m: (jnp.where(p == 1, m, 0), 0, 0)),
            pl.BlockSpec((1, 1, TILE_M),
                         lambda p, m: (jnp.where(p == 1, m, 0), 0, 0)),
            pl.BlockSpec((TILE_M, 4),
                         lambda p, m: (jnp.where(p == 0, m, M_T - 1), 0)),
            pl.BlockSpec((1, NW), lambda p, m: (0, 0)),
            pl.BlockSpec((TILE_M, H // 2),
                         lambda p, m: (jnp.where(p == 0, m, M_T - 1), 0)),
        ],
        out_shape=[
            jax.ShapeDtypeStruct((M_T, 1, TILE_M), jnp.int32),
            jax.ShapeDtypeStruct((M_T, 1, TILE_M), jnp.int32),
            jax.ShapeDtypeStruct((T, 4), jnp.float32),
            jax.ShapeDtypeStruct((1, NW), jnp.int32),
            jax.ShapeDtypeStruct((T, H // 2), jnp.int32),
        ],
        scratch_shapes=[
            pltpu.VMEM((M_T, E), jnp.float32),
            pltpu.VMEM((T, E), jnp.float32),
            pltpu.VMEM((T, E), jnp.float32),
        ],
        compiler_params=pltpu.CompilerParams(
            dimension_semantics=("arbitrary", "arbitrary")),
    )(x32, gate_w, shared_gate_w)


def _make_sc_scatter():
    mesh = plsc.VectorSubcoreMesh(core_axis_name="c", subcore_axis_name="s")

    @functools.partial(
        pl.kernel, mesh=mesh,
        out_type=jax.ShapeDtypeStruct((P_MAX, H // 2), jnp.int32),
        scratch_types=[
            pltpu.VMEM((TPW, H // 2), jnp.int32),
            pltpu.VMEM((TPW,), jnp.int32),
            pltpu.VMEM((TPW,), jnp.int32),
            pltpu.SemaphoreType.DMA,
            pltpu.SemaphoreType.DMA,
        ],
    )
    def sc_scatter(xb_hbm, pos0_hbm, pos1_hbm, xg_hbm, xloc, idx0, idx1,
                   sem0, sem1):
        wid = lax.axis_index("s") * 2 + lax.axis_index("c")
        pltpu.sync_copy(xb_hbm.at[pl.ds(wid * TPW, TPW)], xloc)
        pltpu.sync_copy(pos0_hbm.at[wid, 0], idx0)
        pltpu.sync_copy(pos1_hbm.at[wid, 0], idx1)
        c0 = pltpu.async_copy(xloc, xg_hbm.at[idx0], sem0)
        c1 = pltpu.async_copy(xloc, xg_hbm.at[idx1], sem1)
        c0.wait()
        c1.wait()

    return sc_scatter


def _grouped_kernel(te_ref, xg_ref, wg_ref, wu_ref, wd_ref, yg_ref,
                    wgb, wub, wdb):
    i = pl.program_id(0)
    te = te_ref[i]

    @pl.when(te < E)
    def _work():
        prev = te_ref[jnp.maximum(i - 1, 0)]

        @pl.when((i == 0) | (te != prev))
        def _cast():
            wgb[...] = wg_ref[0].astype(jnp.bfloat16)
            wub[...] = wu_ref[0].astype(jnp.bfloat16)
            wdb[...] = wd_ref[0].astype(jnp.bfloat16)

        xblk = _unpack_bf16(xg_ref[...])
        g = lax.dot_general(xblk, wgb[...], (((1,), (1,)), ((), ())),
                            preferred_element_type=jnp.float32)
        u = lax.dot_general(xblk, wub[...], (((1,), (1,)), ((), ())),
                            preferred_element_type=jnp.float32)
        h = (jax.nn.silu(g) * u).astype(jnp.bfloat16)
        d = lax.dot_general(h, wdb[...], (((1,), (1,)), ((), ())),
                            preferred_element_type=jnp.float32)
        yg_ref[...] = _pack_bf16(d.astype(jnp.bfloat16))


def _run_grouped(tile_e, xg2d, Wg, Wu, Wd):
    grid_spec = pltpu.PrefetchScalarGridSpec(
        num_scalar_prefetch=1,
        grid=(G_TILES,),
        in_specs=[
            pl.BlockSpec((TILE_M, H // 2), lambda i, te: (i, 0)),
            pl.BlockSpec((1, FF, H),
                         lambda i, te: (lax.rem(te[i], E), 0, 0)),
            pl.BlockSpec((1, FF, H),
                         lambda i, te: (lax.rem(te[i], E), 0, 0)),
            pl.BlockSpec((1, H, FF),
                         lambda i, te: (lax.rem(te[i], E), 0, 0)),
        ],
        out_specs=pl.BlockSpec((TILE_M, H // 2), lambda i, te: (i, 0)),
        scratch_shapes=[
            pltpu.VMEM((FF, H), jnp.bfloat16),
            pltpu.VMEM((FF, H), jnp.bfloat16),
            pltpu.VMEM((H, FF), jnp.bfloat16),
        ],
    )
    return pl.pallas_call(
        _grouped_kernel,
        grid_spec=grid_spec,
        out_shape=jax.ShapeDtypeStruct((P_MAX, H // 2), jnp.int32),
        compiler_params=pltpu.CompilerParams(
            dimension_semantics=("arbitrary",),
            vmem_limit_bytes=110 * 1024 * 1024,
        ),
    )(tile_e, xg2d, Wg, Wu, Wd)


def _make_sc_gather():
    mesh = plsc.VectorSubcoreMesh(core_axis_name="c", subcore_axis_name="s")

    @functools.partial(
        pl.kernel, mesh=mesh,
        out_type=[
            jax.ShapeDtypeStruct((T, H // 2), jnp.int32),
            jax.ShapeDtypeStruct((T, H // 2), jnp.int32),
        ],
        scratch_types=[
            pltpu.VMEM((TPW, H // 2), jnp.int32),
            pltpu.VMEM((TPW, H // 2), jnp.int32),
            pltpu.VMEM((TPW,), jnp.int32),
            pltpu.VMEM((TPW,), jnp.int32),
            pltpu.SemaphoreType.DMA,
            pltpu.SemaphoreType.DMA,
        ],
    )
    def sc_gather(yg_hbm, pos0_hbm, pos1_hbm, y0_hbm, y1_hbm, rows0, rows1,
                  idx0, idx1, sem0, sem1):
        wid = lax.axis_index("s") * 2 + lax.axis_index("c")
        pltpu.sync_copy(pos0_hbm.at[wid, 0], idx0)
        pltpu.sync_copy(pos1_hbm.at[wid, 0], idx1)
        c0 = pltpu.async_copy(yg_hbm.at[idx0], rows0, sem0)
        c1 = pltpu.async_copy(yg_hbm.at[idx1], rows1, sem1)
        c0.wait()
        c1.wait()
        pltpu.sync_copy(rows0, y0_hbm.at[pl.ds(wid * TPW, TPW)])
        pltpu.sync_copy(rows1, y1_hbm.at[pl.ds(wid * TPW, TPW)])

    return sc_gather


def _combine_kernel(xb_ref, swg_ref, swu_ref, swd_ref, y0_ref, y1_ref,
                    wc_ref, out_ref, swg_s, swu_s, swd_s):
    m = pl.program_id(0)

    @pl.when(m == 0)
    def _cast():
        swg_s[...] = swg_ref[...].astype(jnp.bfloat16)
        swu_s[...] = swu_ref[...].astype(jnp.bfloat16)
        swd_s[...] = swd_ref[...].astype(jnp.bfloat16)

    xblk = xb_ref[...].astype(jnp.bfloat16)
    g = lax.dot_general(xblk, swg_s[...], (((1,), (1,)), ((), ())),
                        preferred_element_type=jnp.float32)
    u = lax.dot_general(xblk, swu_s[...], (((1,), (1,)), ((), ())),
                        preferred_element_type=jnp.float32)
    h = (jax.nn.silu(g) * u).astype(jnp.bfloat16)
    sh = lax.dot_general(h, swd_s[...], (((1,), (1,)), ((), ())),
                         preferred_element_type=jnp.float32)
    wc = wc_ref[...]
    y0 = _unpack_bf16(y0_ref[...]).astype(jnp.float32)
    y1 = _unpack_bf16(y1_ref[...]).astype(jnp.float32)
    out_ref[...] = wc[:, 0:1] * y0 + wc[:, 1:2] * y1 + wc[:, 2:3] * sh


def _run_combine(xb2d, sWg_f, sWu_f, sWd_f, y0g, y1g, wcomb):
    return pl.pallas_call(
        _combine_kernel,
        grid=(M_T,),
        in_specs=[
            pl.BlockSpec((TILE_M, H), lambda m: (m, 0)),
            pl.BlockSpec((SFF, H), lambda m: (0, 0)),
            pl.BlockSpec((SFF, H), lambda m: (0, 0)),
            pl.BlockSpec((H, SFF), lambda m: (0, 0)),
            pl.BlockSpec((TILE_M, H // 2), lambda m: (m, 0)),
            pl.BlockSpec((TILE_M, H // 2), lambda m: (m, 0)),
            pl.BlockSpec((TILE_M, 4), lambda m: (m, 0)),
        ],
        out_specs=pl.BlockSpec((TILE_M, H), lambda m: (m, 0)),
        out_shape=jax.ShapeDtypeStruct((T, H), jnp.float32),
        scratch_shapes=[
            pltpu.VMEM((SFF, H), jnp.bfloat16),
            pltpu.VMEM((SFF, H), jnp.bfloat16),
            pltpu.VMEM((H, SFF), jnp.bfloat16),
        ],
        compiler_params=pltpu.CompilerParams(
            dimension_semantics=("arbitrary",),
            vmem_limit_bytes=110 * 1024 * 1024,
        ),
    )(xb2d, sWg_f, sWu_f, sWd_f, y0g, y1g, wcomb)


_SC_CACHE = {}


def _sc_scatter():
    if "scatter" not in _SC_CACHE:
        _SC_CACHE["scatter"] = _make_sc_scatter()
    return _SC_CACHE["scatter"]


def _sc_gather():
    if "gather" not in _SC_CACHE:
        _SC_CACHE["gather"] = _make_sc_gather()
    return _SC_CACHE["gather"]


@jax.jit
def kernel(hidden_states, gate_w, Wg, Wu, Wd, sWg, sWu, sWd, shared_gate_w):
    b, s_len, h = hidden_states.shape
    x32 = hidden_states.reshape(T, H)

    pos0, pos1, wcomb, tile_e2, xpk = _run_router(x32, gate_w, shared_gate_w)
    p0w = pos0.reshape(NW, 1, TPW)
    p1w = pos1.reshape(NW, 1, TPW)

    xg = _sc_scatter()(xpk, p0w, p1w)
    yg = _run_grouped(tile_e2.reshape(NW), xg, Wg, Wu, Wd)
    y0g, y1g = _sc_gather()(yg, p0w, p1w)
    out = _run_combine(x32, sWg, sWu, sWd, y0g, y1g, wcomb)
    return out.reshape(b, s_len, h)
